# baseline (speedup 1.0000x reference)
import functools

import jax
import jax.numpy as jnp
from jax import lax
from jax.experimental import pallas as pl
from jax.experimental.pallas import tpu as pltpu
from jax.experimental.pallas import tpu_sc as plsc

MAX_BATCH = 128
BUFFER_LENGTH = 16
HIDDEN_SIZE = 4096

ROWS = MAX_BATCH * BUFFER_LENGTH
NUM_CORES = 2
NUM_SUBCORES = 16
NW = NUM_CORES * NUM_SUBCORES
BPW = MAX_BATCH // NW
RPW = ROWS // NW
LANES = 16

CH_T = 8
NCH_T = 2 * BUFFER_LENGTH // CH_T
DEPTH_T = 3
PRE_T = DEPTH_T - 1

CH_S = 2
NCH_S = 2 * BUFFER_LENGTH // CH_S
DEPTH_S = 3
PRE_S = DEPTH_S - 1


_mesh = plsc.VectorSubcoreMesh(core_axis_name="c", subcore_axis_name="s")


@functools.partial(
    pl.kernel,
    out_type=jax.ShapeDtypeStruct((ROWS, HIDDEN_SIZE), jnp.float32),
    mesh=_mesh,
    compiler_params=pltpu.CompilerParams(needs_layout_passes=False),
    scratch_types=[
        pltpu.VMEM((2 * BPW,), jnp.int32),
        pltpu.VMEM((DEPTH_T, CH_T, HIDDEN_SIZE), jnp.float32),
        pltpu.VMEM_SHARED(
            (NUM_SUBCORES, DEPTH_S, CH_S, HIDDEN_SIZE), jnp.float32),
    ] + [pltpu.SemaphoreType.DMA] * (2 * DEPTH_T + 2 * DEPTH_S),
)
def _sc_update(pos_hbm, hs_hbm, buf_hbm, out_hbm,
               pos_v, tb, sb, *sems):
    cid = lax.axis_index("c")
    sid = lax.axis_index("s")
    wid = sid * NUM_CORES + cid
    r0 = wid * RPW
    b0 = wid * BPW

    pltpu.sync_copy(pos_hbm.at[pl.ds((wid >> 1) * (2 * BPW), 2 * BPW)], pos_v)

    lane = lax.iota(jnp.int32, LANES)
    k4 = lane & (BPW - 1)
    pos16 = plsc.load_gather(pos_v, [(wid & 1) * BPW + k4])
    dl16 = k4 * BUFFER_LENGTH + (pos16 & (BUFFER_LENGTH - 1))
    dl = [jnp.max(jnp.where(lane == k, dl16, -1)) for k in range(BPW)]
    t_chunk = [dl[k] >> 3 for k in range(2)]
    t_row = [dl[k] & (CH_T - 1) for k in range(2)]
    s_chunk = [(dl[k] - 2 * BUFFER_LENGTH) >> 1 for k in (2, 3)]
    s_row = [dl[k] & (CH_S - 1) for k in (2, 3)]

    sem_ti = sems[:DEPTH_T]
    sem_to = sems[DEPTH_T:2 * DEPTH_T]
    sem_si = sems[2 * DEPTH_T:2 * DEPTH_T + DEPTH_S]
    sem_so = sems[2 * DEPTH_T + DEPTH_S:]
    t_in = [
        pltpu.make_async_copy(
            buf_hbm.at[pl.ds(r0 + c * CH_T, CH_T)], tb.at[c % DEPTH_T],
            sem_ti[c % DEPTH_T])
        for c in range(NCH_T)
    ]
    t_out = [
        pltpu.make_async_copy(
            tb.at[c % DEPTH_T], out_hbm.at[pl.ds(r0 + c * CH_T, CH_T)],
            sem_to[c % DEPTH_T])
        for c in range(NCH_T)
    ]
    s0 = r0 + 2 * BUFFER_LENGTH
    s_in = [
        pltpu.make_async_copy(
            buf_hbm.at[pl.ds(s0 + c * CH_S, CH_S)], sb.at[sid].at[c % DEPTH_S],
            sem_si[c % DEPTH_S])
        for c in range(NCH_S)
    ]
    s_out = [
        pltpu.make_async_copy(
            sb.at[sid].at[c % DEPTH_S], out_hbm.at[pl.ds(s0 + c * CH_S, CH_S)],
            sem_so[c % DEPTH_S])
        for c in range(NCH_S)
    ]

    for c in range(PRE_T):
        t_in[c].start()
    for c in range(PRE_S):
        s_in[c].start()

    for c in range(NCH_S):
        if c < NCH_T:
            t_in[c].wait()
            k = c // 2
            @pl.when(t_chunk[k] == c)
            def _():
                pltpu.sync_copy(
                    hs_hbm.at[pl.ds(b0 + k, 1)],
                    tb.at[c % DEPTH_T].at[pl.ds(t_row[k], 1)])
            t_out[c].start()
            if c + PRE_T < NCH_T:
                if c >= 1:
                    t_out[c - 1].wait()
                t_in[c + PRE_T].start()
        s_in[c].wait()
        k = c // 8
        @pl.when(s_chunk[k] == c)
        def _():
            pltpu.sync_copy(
                hs_hbm.at[pl.ds(b0 + 2 + k, 1)],
                sb.at[sid].at[c % DEPTH_S].at[pl.ds(s_row[k], 1)])
        s_out[c].start()
        if c + PRE_S < NCH_S:
            if c >= 1:
                s_out[c - 1].wait()
            s_in[c + PRE_S].start()

    for c in range(NCH_T - DEPTH_T, NCH_T):
        t_out[c].wait()
    for c in range(NCH_S - DEPTH_S, NCH_S):
        s_out[c].wait()


def kernel(seq_ids, position_ids, hidden_state, hidden_states):
    del seq_ids
    pos = position_ids.reshape(MAX_BATCH).astype(jnp.int32)
    hs2d = hidden_state.reshape(MAX_BATCH, HIDDEN_SIZE)
    buf2d = hidden_states.reshape(ROWS, HIDDEN_SIZE)
    out2d = _sc_update(pos, hs2d, buf2d)
    return out2d.reshape(MAX_BATCH, BUFFER_LENGTH, HIDDEN_SIZE)


# speedup vs baseline: 1.0236x; 1.0236x over previous
import functools

import jax
import jax.numpy as jnp
from jax import lax
from jax.experimental import pallas as pl
from jax.experimental.pallas import tpu as pltpu
from jax.experimental.pallas import tpu_sc as plsc

MAX_BATCH = 128
BUFFER_LENGTH = 16
HIDDEN_SIZE = 4096

ROWS = MAX_BATCH * BUFFER_LENGTH
NUM_CORES = 2
NUM_SUBCORES = 16
NW = NUM_CORES * NUM_SUBCORES
BPW = MAX_BATCH // NW
RPW = ROWS // NW
LANES = 16

TB = 3
CH_T = 8
NCH_T = TB * BUFFER_LENGTH // CH_T
DEPTH_T = 3
PRE_T = DEPTH_T - 1

CH_S = 2
NCH_S = (BPW - TB) * BUFFER_LENGTH // CH_S
DEPTH_S = 3
PRE_S = DEPTH_S - 1


_mesh = plsc.VectorSubcoreMesh(core_axis_name="c", subcore_axis_name="s")


@functools.partial(
    pl.kernel,
    out_type=jax.ShapeDtypeStruct((ROWS, HIDDEN_SIZE), jnp.float32),
    mesh=_mesh,
    compiler_params=pltpu.CompilerParams(needs_layout_passes=False),
    scratch_types=[
        pltpu.VMEM((2 * BPW,), jnp.int32),
        pltpu.VMEM((DEPTH_T, CH_T, HIDDEN_SIZE), jnp.float32),
        pltpu.VMEM_SHARED(
            (NUM_SUBCORES, DEPTH_S, CH_S, HIDDEN_SIZE), jnp.float32),
    ] + [pltpu.SemaphoreType.DMA] * (2 * DEPTH_T + 2 * DEPTH_S),
)
def _sc_update(pos_hbm, hs_hbm, buf_hbm, out_hbm,
               pos_v, tb, sb, *sems):
    cid = lax.axis_index("c")
    sid = lax.axis_index("s")
    wid = sid * NUM_CORES + cid
    r0 = wid * RPW
    b0 = wid * BPW

    pltpu.sync_copy(pos_hbm.at[pl.ds((wid >> 1) * (2 * BPW), 2 * BPW)], pos_v)

    lane = lax.iota(jnp.int32, LANES)
    k4 = lane & (BPW - 1)
    pos16 = plsc.load_gather(pos_v, [(wid & 1) * BPW + k4])
    dl16 = k4 * BUFFER_LENGTH + (pos16 & (BUFFER_LENGTH - 1))
    dl = [jnp.max(jnp.where(lane == k, dl16, -1)) for k in range(BPW)]
    t_chunk = [dl[k] >> 3 for k in range(TB)]
    t_row = [dl[k] & (CH_T - 1) for k in range(TB)]
    s_chunk = [(dl[k] - TB * BUFFER_LENGTH) >> 1 for k in range(TB, BPW)]
    s_row = [dl[k] & (CH_S - 1) for k in range(TB, BPW)]

    sem_ti = sems[:DEPTH_T]
    sem_to = sems[DEPTH_T:2 * DEPTH_T]
    sem_si = sems[2 * DEPTH_T:2 * DEPTH_T + DEPTH_S]
    sem_so = sems[2 * DEPTH_T + DEPTH_S:]
    t_in = [
        pltpu.make_async_copy(
            buf_hbm.at[pl.ds(r0 + c * CH_T, CH_T)], tb.at[c % DEPTH_T],
            sem_ti[c % DEPTH_T])
        for c in range(NCH_T)
    ]
    t_out = [
        pltpu.make_async_copy(
            tb.at[c % DEPTH_T], out_hbm.at[pl.ds(r0 + c * CH_T, CH_T)],
            sem_to[c % DEPTH_T])
        for c in range(NCH_T)
    ]
    s0 = r0 + TB * BUFFER_LENGTH
    s_in = [
        pltpu.make_async_copy(
            buf_hbm.at[pl.ds(s0 + c * CH_S, CH_S)], sb.at[sid].at[c % DEPTH_S],
            sem_si[c % DEPTH_S])
        for c in range(NCH_S)
    ]
    s_out = [
        pltpu.make_async_copy(
            sb.at[sid].at[c % DEPTH_S], out_hbm.at[pl.ds(s0 + c * CH_S, CH_S)],
            sem_so[c % DEPTH_S])
        for c in range(NCH_S)
    ]

    for c in range(PRE_T):
        t_in[c].start()
    for c in range(PRE_S):
        s_in[c].start()

    for c in range(max(NCH_S, NCH_T)):
        if c < NCH_T:
            t_in[c].wait()
            k = c // 2
            @pl.when(t_chunk[k] == c)
            def _():
                pltpu.sync_copy(
                    hs_hbm.at[pl.ds(b0 + k, 1)],
                    tb.at[c % DEPTH_T].at[pl.ds(t_row[k], 1)])
            t_out[c].start()
            if c + PRE_T < NCH_T:
                if c >= 1:
                    t_out[c - 1].wait()
                t_in[c + PRE_T].start()
        if c < NCH_S:
            s_in[c].wait()
            k = c // (BUFFER_LENGTH // CH_S)
            @pl.when(s_chunk[k] == c)
            def _():
                pltpu.sync_copy(
                    hs_hbm.at[pl.ds(b0 + TB + k, 1)],
                    sb.at[sid].at[c % DEPTH_S].at[pl.ds(s_row[k], 1)])
            s_out[c].start()
            if c + PRE_S < NCH_S:
                if c >= 1:
                    s_out[c - 1].wait()
                s_in[c + PRE_S].start()

    for c in range(NCH_T - DEPTH_T, NCH_T):
        t_out[c].wait()
    for c in range(NCH_S - DEPTH_S, NCH_S):
        s_out[c].wait()


def kernel(seq_ids, position_ids, hidden_state, hidden_states):
    del seq_ids
    pos = position_ids.reshape(MAX_BATCH).astype(jnp.int32)
    hs2d = hidden_state.reshape(MAX_BATCH, HIDDEN_SIZE)
    buf2d = hidden_states.reshape(ROWS, HIDDEN_SIZE)
    out2d = _sc_update(pos, hs2d, buf2d)
    return out2d.reshape(MAX_BATCH, BUFFER_LENGTH, HIDDEN_SIZE)


# speedup vs baseline: 1.0266x; 1.0029x over previous
import functools

import jax
import jax.numpy as jnp
from jax import lax
from jax.experimental import pallas as pl
from jax.experimental.pallas import tpu as pltpu
from jax.experimental.pallas import tpu_sc as plsc

MAX_BATCH = 128
BUFFER_LENGTH = 16
HIDDEN_SIZE = 4096

ROWS = MAX_BATCH * BUFFER_LENGTH
NUM_CORES = 2
NUM_SUBCORES = 16
NW = NUM_CORES * NUM_SUBCORES
BPW = MAX_BATCH // NW
RPW = ROWS // NW
CH = 8
NCH = RPW // CH
DEPTH = 3
PRE = DEPTH - 1
CPB = BUFFER_LENGTH // CH
LANES = 16


_mesh = plsc.VectorSubcoreMesh(core_axis_name="c", subcore_axis_name="s")


@functools.partial(
    pl.kernel,
    out_type=jax.ShapeDtypeStruct((ROWS, HIDDEN_SIZE), jnp.float32),
    mesh=_mesh,
    compiler_params=pltpu.CompilerParams(needs_layout_passes=False, use_tc_tiling_on_sc=True),
    scratch_types=[
        pltpu.VMEM((2 * BPW,), jnp.int32),
        pltpu.VMEM((DEPTH, CH, HIDDEN_SIZE), jnp.float32),
        pltpu.VMEM_SHARED((NUM_SUBCORES * BPW, HIDDEN_SIZE), jnp.float32),
        pltpu.SemaphoreType.DMA,
    ] + [pltpu.SemaphoreType.DMA] * (2 * DEPTH),
)
def _sc_update(pos_hbm, hs_hbm, buf_hbm, out_hbm,
               pos_v, db, hs_sp, sem_hs, *sems):
    cid = lax.axis_index("c")
    sid = lax.axis_index("s")
    wid = sid * NUM_CORES + cid
    r0 = wid * RPW
    b0 = wid * BPW

    sp0 = sid * BPW
    cp_hs = pltpu.make_async_copy(
        hs_hbm.at[pl.ds(b0, BPW)], hs_sp.at[pl.ds(sp0, BPW)], sem_hs)
    cp_hs.start()

    pltpu.sync_copy(pos_hbm.at[pl.ds((wid >> 1) * (2 * BPW), 2 * BPW)], pos_v)

    lane = lax.iota(jnp.int32, LANES)
    k4 = lane & (BPW - 1)
    pos16 = plsc.load_gather(pos_v, [(wid & 1) * BPW + k4])
    dl16 = k4 * BUFFER_LENGTH + (pos16 & (BUFFER_LENGTH - 1))
    chunk_shift = CH.bit_length() - 1
    chunk_of = []
    row_of = []
    for k in range(BPW):
        dlk = jnp.max(jnp.where(lane == k, dl16, -1))
        chunk_of.append(dlk >> chunk_shift)
        row_of.append(dlk & (CH - 1))

    sem_in = sems[:DEPTH]
    sem_out = sems[DEPTH:]
    cp_in = [
        pltpu.make_async_copy(
            buf_hbm.at[pl.ds(r0 + c * CH, CH)], db.at[c % DEPTH],
            sem_in[c % DEPTH])
        for c in range(NCH)
    ]
    cp_out = [
        pltpu.make_async_copy(
            db.at[c % DEPTH], out_hbm.at[pl.ds(r0 + c * CH, CH)],
            sem_out[c % DEPTH])
        for c in range(NCH)
    ]
    for c in range(PRE):
        cp_in[c].start()
    cp_hs.wait()
    for c in range(NCH):
        cp_in[c].wait()
        k = c // CPB
        @pl.when(chunk_of[k] == c)
        def _():
            pltpu.sync_copy(hs_sp.at[pl.ds(sp0 + k, 1)],
                            db.at[c % DEPTH].at[pl.ds(row_of[k], 1)])
        cp_out[c].start()
        if c + PRE < NCH:
            if c >= 1:
                cp_out[c - 1].wait()
            cp_in[c + PRE].start()
    for c in range(NCH - DEPTH, NCH):
        cp_out[c].wait()


def kernel(seq_ids, position_ids, hidden_state, hidden_states):
    del seq_ids
    pos = position_ids.reshape(MAX_BATCH).astype(jnp.int32)
    hs2d = hidden_state.reshape(MAX_BATCH, HIDDEN_SIZE)
    buf2d = hidden_states.reshape(ROWS, HIDDEN_SIZE)
    out2d = _sc_update(pos, hs2d, buf2d)
    return out2d.reshape(MAX_BATCH, BUFFER_LENGTH, HIDDEN_SIZE)
